# Initial kernel scaffold; baseline (speedup 1.0000x reference)
import jax
import jax.numpy as jnp
from jax.experimental import pallas as pl


def kernel(words, word_table):
    raise NotImplementedError("write your pallas kernel here")


# speedup vs baseline: 1.4874x; 1.4874x over previous
import functools

import jax
import jax.numpy as jnp
from jax import lax
from jax.experimental import pallas as pl
from jax.experimental.pallas import tpu as pltpu
from jax.experimental.pallas import tpu_sc as plsc

NC = 2
NS = 16
NW = NC * NS

B = 4096 * 200
D = 32
BPW = B // NW

G = 128
K = 8
CHUNK = K * G
NCHUNKS = BPW // CHUNK
IDX_ROWS_PER_W = BPW // G

_mesh = plsc.VectorSubcoreMesh(core_axis_name="c", subcore_axis_name="s")


@functools.partial(
    pl.kernel,
    mesh=_mesh,
    compiler_params=pltpu.CompilerParams(use_tc_tiling_on_sc=False),
    out_type=jax.ShapeDtypeStruct((B, D), jnp.float32),
    scratch_types=[
        pltpu.VMEM((2, K, G), jnp.int32),
        pltpu.VMEM((2, CHUNK, D), jnp.float32),
        pltpu.SemaphoreType.DMA,
        pltpu.SemaphoreType.DMA,
    ],
)
def _sc_gather(idx_hbm, table_hbm, out_hbm, idx_v, rows_v, gsem0, gsem1):
    wid = lax.axis_index("s") * NC + lax.axis_index("c")
    idx_row0 = wid * IDX_ROWS_PER_W
    out_row0 = wid * BPW
    gsems = (gsem0, gsem1)

    def load_idx(j, s):
        pltpu.sync_copy(idx_hbm.at[pl.ds(idx_row0 + j * K, K)], idx_v.at[s])

    def start_gather(s):
        for r in range(K):
            pltpu.async_copy(
                table_hbm.at[idx_v.at[s, r]],
                rows_v.at[s, pl.ds(r * G, G)],
                gsems[s],
            )

    def wait_gather(s):
        pltpu.make_async_copy(
            out_hbm.at[pl.ds(0, CHUNK)], rows_v.at[s], gsems[s]
        ).wait()

    def write_out(j, s):
        pltpu.sync_copy(rows_v.at[s], out_hbm.at[pl.ds(out_row0 + j * CHUNK, CHUNK)])

    load_idx(0, 0)
    start_gather(0)

    @pl.loop(1, NCHUNKS - 1, step=2)
    def _(g):
        load_idx(g, 1)
        start_gather(1)
        wait_gather(0)
        write_out(g - 1, 0)
        load_idx(g + 1, 0)
        start_gather(0)
        wait_gather(1)
        write_out(g, 1)

    wait_gather(0)
    write_out(NCHUNKS - 1, 0)


def kernel(words, word_table):
    flat_idx = words.reshape(B // G, G)
    out = _sc_gather(flat_idx, word_table)
    return out.reshape(*words.shape, D)
